# Initial kernel scaffold; baseline (speedup 1.0000x reference)
import jax
import jax.numpy as jnp
from jax.experimental import pallas as pl


def kernel(l_input_features, h_inputs_features, candidate_windows_mask, coords_list, W_csf):
    raise NotImplementedError("write your pallas kernel here")


# speedup vs baseline: 67.6283x; 67.6283x over previous
import jax
import jax.numpy as jnp
from jax import lax
from jax.experimental import pallas as pl
from jax.experimental.pallas import tpu as pltpu

_N, _C, _H, _W = 256, 96, 32, 32
_B, _G = 4, 64
_HW = _H * _W
_NB = 8
_NCELL = _B * 64
_NBAND = _B * 8
_NSTEP = _N + _NBAND


def _csf_body(wl_ref, wh_ref, l_ref, h_ref, out_ref):
    for j in range(_NB):
        acc = lax.dot_general(wl_ref[...], l_ref[j], (((1,), (0,)), ((), ())),
                              preferred_element_type=jnp.float32)
        acc = acc + lax.dot_general(wh_ref[...], h_ref[j], (((1,), (0,)), ((), ())),
                                    preferred_element_type=jnp.float32)
        out_ref[j] = jnp.maximum(acc, 0.0)


def _csf(l3, h3, W_csf):
    wl = W_csf[:_C].T
    wh = W_csf[_C:].T
    return pl.pallas_call(
        _csf_body,
        grid=(_N // _NB,),
        in_specs=[
            pl.BlockSpec((_C, _C), lambda i: (0, 0)),
            pl.BlockSpec((_C, _C), lambda i: (0, 0)),
            pl.BlockSpec((_NB, _C, _HW), lambda i: (i, 0, 0)),
            pl.BlockSpec((_NB, _C, _HW), lambda i: (i, 0, 0)),
        ],
        out_specs=pl.BlockSpec((_NB, _C, _HW), lambda i: (i, 0, 0)),
        out_shape=jax.ShapeDtypeStruct((_N, _C, _HW), jnp.float32),
    )(wl, wh, l3, h3)


def _scatter_body(sb, sgy, sgx, sfetch, sinit, scnt, pred_ref, out_ref):
    s = pl.program_id(0)

    @pl.when(sinit[s] == 1)
    def _zero():
        out_ref[...] = jnp.zeros_like(out_ref)

    @pl.when(sinit[s] == 0)
    def _acc():
        scale = 1.0 / (scnt[s].astype(jnp.float32) + 1e-6)
        val = pred_ref[0] * scale
        gx = sgx[s]
        for g in range(8):
            @pl.when(gx == g)
            def _add(g=g):
                out_ref[0, :, :, g * _W:(g + 1) * _W] = (
                    out_ref[0, :, :, g * _W:(g + 1) * _W] + val)


def _schedule(coords):
    n_idx = jnp.arange(_N, dtype=jnp.int32)
    cell = (n_idx // _G) * 64 + coords[:, 0] * 8 + coords[:, 1]
    counts = jnp.zeros((_NCELL,), jnp.int32).at[cell].add(1)
    order = jnp.argsort(cell).astype(jnp.int32)
    cell_sorted = cell[order]
    band_counts = counts.reshape(_NBAND, 8).sum(axis=1)
    bsum = jnp.cumsum(band_counts) - band_counts
    band_idx = jnp.arange(_NBAND, dtype=jnp.int32)
    run_start = band_idx + bsum
    win_steps = cell_sorted // 8 + 1 + jnp.arange(_N, dtype=jnp.int32)
    step_fetch = jnp.zeros((_NSTEP,), jnp.int32).at[win_steps].set(order)
    first_win = jnp.where(band_counts > 0,
                          order[jnp.minimum(bsum, _N - 1)], 0)
    step_fetch = step_fetch.at[run_start].set(first_win)
    is_init = jnp.zeros((_NSTEP,), jnp.int32).at[run_start].set(1)
    step_band = jnp.repeat(band_idx, band_counts + 1,
                           total_repeat_length=_NSTEP)
    step_b = step_band // 8
    step_gy = step_band % 8
    step_gx = jnp.zeros((_NSTEP,), jnp.int32).at[win_steps].set(cell_sorted % 8)
    step_cnt = jnp.ones((_NSTEP,), jnp.int32).at[win_steps].set(
        counts[cell_sorted])
    return step_b, step_gy, step_gx, step_fetch, is_init, step_cnt


def _stitch(preds4, coords):
    step_b, step_gy, step_gx, step_fetch, is_init, step_cnt = _schedule(coords)
    grid_spec = pltpu.PrefetchScalarGridSpec(
        num_scalar_prefetch=6,
        grid=(_NSTEP,),
        in_specs=[
            pl.BlockSpec((1, _C, _H, _W),
                         lambda s, sb, sgy, sgx, sf, si, sc: (sf[s], 0, 0, 0)),
        ],
        out_specs=pl.BlockSpec((1, _C, _H, _W * 8),
                               lambda s, sb, sgy, sgx, sf, si, sc:
                               (sb[s], 0, sgy[s], 0)),
    )
    return pl.pallas_call(
        _scatter_body,
        grid_spec=grid_spec,
        out_shape=jax.ShapeDtypeStruct((_B, _C, _H * 8, _W * 8), jnp.float32),
    )(step_b, step_gy, step_gx, step_fetch, is_init, step_cnt, preds4)


def kernel(l_input_features, h_inputs_features, candidate_windows_mask,
           coords_list, W_csf):
    l3 = l_input_features.reshape(_N, _C, _HW)
    h3 = h_inputs_features.reshape(_N, _C, _HW)
    preds = _csf(l3, h3, W_csf)
    full = _stitch(preds.reshape(_N, _C, _H, _W), coords_list)
    return full, preds.reshape(_N, _C, _H, _W)
